# baseline (speedup 1.0000x reference)
import jax
import jax.numpy as jnp
from jax.experimental import pallas as pl
from jax.experimental.pallas import tpu as pltpu

INPUT_MIN, INPUT_MAX = 0.0, 1.0

_KC = 8


def _spline_body(x_ref, w_ref, bias_ref, out_ref, idx_ref, xbf_ref):
    g = pl.program_id(0)
    num_g = pl.num_programs(0)
    in_dim = x_ref.shape[1]
    kc = w_ref.shape[0]
    num_k = kc * num_g

    @pl.when(g == 0)
    def _init():
        xv = x_ref[:]
        x_norm = (xv - INPUT_MIN) / (INPUT_MAX - INPUT_MIN)
        idx_ref[:] = jnp.clip(
            jnp.floor(x_norm * num_k), 0.0, num_k - 1.0).astype(jnp.bfloat16)
        xbf_ref[:] = xv.astype(jnp.bfloat16)

    idx = idx_ref[:]
    xbf = xbf_ref[:]
    acc = jnp.zeros((x_ref.shape[0], w_ref.shape[2]), jnp.float32)
    base = g * kc
    for kk in range(kc):
        sel = idx == (base + kk).astype(jnp.bfloat16)
        xm = jnp.where(sel, xbf, jnp.bfloat16(0))
        mask = jnp.where(sel, jnp.bfloat16(1), jnp.bfloat16(0))
        acc = acc + jnp.dot(xm, w_ref[kk, :in_dim, :],
                            preferred_element_type=jnp.float32)
        acc = acc + jnp.dot(mask, w_ref[kk, in_dim:, :],
                            preferred_element_type=jnp.float32)

    @pl.when(g == 0)
    def _first():
        out_ref[:] = acc + bias_ref[:]

    @pl.when(g != 0)
    def _rest():
        out_ref[:] = out_ref[:] + acc


def kernel(x, slopes, intercepts, bias):
    b, in_dim = x.shape
    out_dim, _, k = slopes.shape
    s_t = jnp.transpose(slopes, (2, 1, 0))
    t_t = jnp.transpose(intercepts, (2, 1, 0))
    w = jnp.concatenate([s_t, t_t], axis=1).astype(jnp.bfloat16)
    bias2d = bias.reshape(1, out_dim)

    return pl.pallas_call(
        _spline_body,
        grid=(k // _KC,),
        in_specs=[
            pl.BlockSpec((b, in_dim), lambda g: (0, 0)),
            pl.BlockSpec((_KC, 2 * in_dim, out_dim), lambda g: (g, 0, 0)),
            pl.BlockSpec((1, out_dim), lambda g: (0, 0)),
        ],
        out_specs=pl.BlockSpec((b, out_dim), lambda g: (0, 0)),
        out_shape=jax.ShapeDtypeStruct((b, out_dim), jnp.float32),
        scratch_shapes=[
            pltpu.VMEM((b, in_dim), jnp.bfloat16),
            pltpu.VMEM((b, in_dim), jnp.bfloat16),
        ],
    )(x, w, bias2d)
